# Initial kernel scaffold; baseline (speedup 1.0000x reference)
import jax
import jax.numpy as jnp
from jax.experimental import pallas as pl


def kernel(x, edge_attr, c1_nn_w, c1_nn_b, c1_root_w, c1_root_b, c2_nn_w, c2_nn_b, c2_root_w, c2_root_b, c4_nn_w, c4_nn_b, c4_root_w, c4_root_b, bn1_g, bn1_b, bn2_g, bn2_b, bn3_g, bn3_b, bn4_g, bn4_b, gat_w, gat_asrc, gat_adst, gat_b, edge_index, batch_vector, mask):
    raise NotImplementedError("write your pallas kernel here")


# speedup vs baseline: 2.1470x; 2.1470x over previous
import functools
import jax
import jax.numpy as jnp
import numpy as np
from jax.experimental import pallas as pl
from jax.experimental.pallas import tpu as pltpu

N = 10000
E = 160000
D = 16
B = 64
EB = 2000

_R16 = np.kron(np.eye(16, dtype=np.float32), np.ones((1, 16), np.float32))
_S16 = np.kron(np.ones((16, 1), np.float32), np.eye(16, dtype=np.float32))


def _edge_body(ea_ref, xs_ref, w_ref, b_ref, r_ref, s_ref, msg_ref):
    theta = jnp.dot(ea_ref[...], w_ref[...], preferred_element_type=jnp.float32)
    theta = jnp.maximum(theta + b_ref[...], 0.0)
    xs2 = jnp.dot(xs_ref[...], r_ref[...], preferred_element_type=jnp.float32)
    msg_ref[...] = jnp.dot(xs2 * theta, s_ref[...],
                           preferred_element_type=jnp.float32)


def _edge_msgs(ea, xs, nn_w, nn_b):
    grid = E // EB
    return pl.pallas_call(
        _edge_body,
        grid=(grid,),
        in_specs=[
            pl.BlockSpec((EB, D), lambda i: (i, 0)),
            pl.BlockSpec((EB, D), lambda i: (i, 0)),
            pl.BlockSpec((D, D * D), lambda i: (0, 0)),
            pl.BlockSpec((1, D * D), lambda i: (0, 0)),
            pl.BlockSpec((D, D * D), lambda i: (0, 0)),
            pl.BlockSpec((D * D, D), lambda i: (0, 0)),
        ],
        out_specs=pl.BlockSpec((EB, D), lambda i: (i, 0)),
        out_shape=jax.ShapeDtypeStruct((E, D), jnp.float32),
    )(ea, xs, nn_w, nn_b.reshape(1, -1), jnp.asarray(_R16), jnp.asarray(_S16))


def _node_body(s_ref, c_ref, x_ref, rw_ref, rb_ref, g_ref, b_ref, h_ref):
    agg = s_ref[...] / jnp.maximum(c_ref[...], 1.0)
    pre = agg + jnp.dot(x_ref[...], rw_ref[...],
                        preferred_element_type=jnp.float32) + rb_ref[...]
    mu = jnp.mean(pre, axis=0, keepdims=True)
    var = jnp.mean((pre - mu) * (pre - mu), axis=0, keepdims=True)
    h_ref[...] = (pre - mu) * jax.lax.rsqrt(var + 1e-5) * g_ref[...] + b_ref[...]


def _node_update(msum, cnt, x, root_w, root_b, g, b):
    return pl.pallas_call(
        _node_body,
        out_shape=jax.ShapeDtypeStruct((N, D), jnp.float32),
    )(msum, cnt, x, root_w, root_b.reshape(1, -1), g.reshape(1, -1),
      b.reshape(1, -1))


def _gat_edge_body(hs_ref, hd_ref, as_ref, ad_ref, out_ref):
    s = jnp.dot(hs_ref[...], as_ref[...], preferred_element_type=jnp.float32)
    t = jnp.dot(hd_ref[...], ad_ref[...], preferred_element_type=jnp.float32)
    e = s + t
    e = jnp.where(e >= 0, e, 0.2 * e)
    ex = jnp.exp(e)
    out_ref[:, :D] = hs_ref[...] * ex
    out_ref[:, D:] = jnp.broadcast_to(ex, (EB, D))


def _gat_edge(hs, hd, a_src, a_dst):
    grid = E // EB
    return pl.pallas_call(
        _gat_edge_body,
        grid=(grid,),
        in_specs=[
            pl.BlockSpec((EB, D), lambda i: (i, 0)),
            pl.BlockSpec((EB, D), lambda i: (i, 0)),
            pl.BlockSpec((D, 1), lambda i: (0, 0)),
            pl.BlockSpec((D, 1), lambda i: (0, 0)),
        ],
        out_specs=pl.BlockSpec((EB, 2 * D), lambda i: (i, 0)),
        out_shape=jax.ShapeDtypeStruct((E, 2 * D), jnp.float32),
    )(hs, hd, a_src.reshape(-1, 1), a_dst.reshape(-1, 1))


def _gat_node_body(nd_ref, bias_ref, g_ref, b_ref, h_ref):
    num = nd_ref[:, :D]
    den = nd_ref[:, D:]
    pre = num / jnp.maximum(den, 1e-16) + bias_ref[...]
    mu = jnp.mean(pre, axis=0, keepdims=True)
    var = jnp.mean((pre - mu) * (pre - mu), axis=0, keepdims=True)
    h_ref[...] = (pre - mu) * jax.lax.rsqrt(var + 1e-5) * g_ref[...] + b_ref[...]


def _gat_node(numden, bias, g, b):
    return pl.pallas_call(
        _gat_node_body,
        out_shape=jax.ShapeDtypeStruct((N, D), jnp.float32),
    )(numden, bias.reshape(1, -1), g.reshape(1, -1), b.reshape(1, -1))


def _matmul_body(x_ref, w_ref, o_ref):
    o_ref[...] = jnp.dot(x_ref[...], w_ref[...],
                         preferred_element_type=jnp.float32)


def _matmul(x, w):
    return pl.pallas_call(
        _matmul_body,
        out_shape=jax.ShapeDtypeStruct((x.shape[0], w.shape[1]), jnp.float32),
    )(x, w)


def _pool_body(hp_ref, bvp_ref, out_ref):
    hp = hp_ref[...]
    bvp = bvp_ref[...]
    neg = jnp.float32(-3.4e38)
    srows, crows, mrows = [], [], []
    for i in range(B):
        msk = bvp == i
        srows.append(jnp.sum(jnp.where(msk, hp, 0.0), axis=0, keepdims=True))
        crows.append(jnp.sum(jnp.where(msk, 1.0, 0.0), axis=0, keepdims=True))
        mrows.append(jnp.max(jnp.where(msk, hp, neg), axis=0, keepdims=True))
    sm = jnp.concatenate(srows, axis=0)
    cm = jnp.concatenate(crows, axis=0)
    mm = jnp.concatenate(mrows, axis=0)
    ssum = sm[:, 0:D]
    csum = cm[:, 0:D]
    mmax = mm[:, 0:D]
    for g in range(1, 8):
        ssum = ssum + sm[:, g * D:(g + 1) * D]
        csum = csum + cm[:, g * D:(g + 1) * D]
        mmax = jnp.maximum(mmax, mm[:, g * D:(g + 1) * D])
    cnt = csum
    mean = ssum / jnp.maximum(cnt, 1.0)
    mmax = jnp.where(cnt > 0, mmax, 0.0)
    out_ref[...] = jnp.concatenate([mean, mmax], axis=1)


def _pool(h, bv):
    hp = h.reshape(N // 8, 8 * D)
    bvp = jnp.repeat(bv, D).reshape(N // 8, 8 * D)
    return pl.pallas_call(
        _pool_body,
        out_shape=jax.ShapeDtypeStruct((B, 2 * D), jnp.float32),
    )(hp, bvp)


def _gather_rows(table, idx):
    return table[idx]


def _scatter_add(vals, dst, width):
    return jax.ops.segment_sum(vals, dst, num_segments=N)


def kernel(x, edge_attr, c1_nn_w, c1_nn_b, c1_root_w, c1_root_b, c2_nn_w,
           c2_nn_b, c2_root_w, c2_root_b, c4_nn_w, c4_nn_b, c4_root_w,
           c4_root_b, bn1_g, bn1_b, bn2_g, bn2_b, bn3_g, bn3_b, bn4_g, bn4_b,
           gat_w, gat_asrc, gat_adst, gat_b, edge_index, batch_vector, mask):
    src = edge_index[0]
    dst = edge_index[1]

    xs = _gather_rows(x, src)
    msg1 = _edge_msgs(edge_attr, xs, c1_nn_w, c1_nn_b)
    m1ones = jnp.concatenate([msg1, jnp.ones((E, D), jnp.float32)], axis=1)
    sc1 = _scatter_add(m1ones, dst, 2 * D)
    msum1, cnt = sc1[:, :D], sc1[:, D:]
    h1 = _node_update(msum1, cnt, x, c1_root_w, c1_root_b, bn1_g, bn1_b)

    hs1 = _gather_rows(h1, src)
    msg2 = _edge_msgs(edge_attr, hs1, c2_nn_w, c2_nn_b)
    msum2 = _scatter_add(msg2, dst, D)
    h2 = _node_update(msum2, cnt, h1, c2_root_w, c2_root_b, bn2_g, bn2_b)

    hw = _matmul(h2, gat_w)
    hws = _gather_rows(hw, src)
    hwd = _gather_rows(hw, dst)
    wex = _gat_edge(hws, hwd, gat_asrc, gat_adst)
    numden = _scatter_add(wex, dst, 2 * D)
    h3 = _gat_node(numden, gat_b, bn3_g, bn3_b)

    hs3 = _gather_rows(h3, src)
    msg4 = _edge_msgs(edge_attr, hs3, c4_nn_w, c4_nn_b)
    msum4 = _scatter_add(msg4, dst, D)
    h4 = _node_update(msum4, cnt, h3, c4_root_w, c4_root_b, bn4_g, bn4_b)

    return _pool(h4, batch_vector)


# speedup vs baseline: 5.1084x; 2.3793x over previous
import functools
import jax
import jax.numpy as jnp
import numpy as np
from jax import lax
from jax.experimental import pallas as pl
from jax.experimental.pallas import tpu as pltpu
from jax.experimental.pallas import tpu_sc as plsc

N = 10000
E = 160000
D = 16
B = 64

NC = 2
NS = 16
NW = NC * NS
CHUNK = 128
CPT = 40
TPT = CPT * CHUNK
EP = NW * TPT
NSTRIPE = N // NS

EB = 1280
GRID_EP = EP // EB

_R16 = np.kron(np.eye(16, dtype=np.float32), np.ones((1, 16), np.float32))
_S16 = np.kron(np.ones((16, 1), np.float32), np.eye(16, dtype=np.float32))


def _sc_gather(table, idx3, sections):
    scpt = sections * CPT
    mesh = plsc.VectorSubcoreMesh(core_axis_name="c", subcore_axis_name="s")

    def body(table_hbm, idx_hbm, out_hbm, idx_v, rows_v, sem):
        wid = lax.axis_index("s") * NC + lax.axis_index("c")
        pltpu.sync_copy(idx_hbm.at[wid], idx_v)
        for sec in range(sections):
            copies = []
            for j in range(CPT):
                copies.append(pltpu.async_copy(
                    table_hbm.at[idx_v.at[sec * CPT + j]],
                    rows_v.at[pl.ds(j * CHUNK, CHUNK)], sem))
            for c in copies:
                c.wait()
            pltpu.sync_copy(rows_v,
                            out_hbm.at[pl.ds(sec * EP + wid * TPT, TPT)])

    f = pl.kernel(
        body,
        out_type=jax.ShapeDtypeStruct((sections * EP, D), jnp.float32),
        mesh=mesh,
        compiler_params=pltpu.CompilerParams(use_tc_tiling_on_sc=False),
        scratch_types=[
            pltpu.VMEM((scpt, CHUNK), jnp.int32),
            pltpu.VMEM((TPT, D), jnp.float32),
            pltpu.SemaphoreType.DMA,
        ],
    )
    return f(table, idx3)


def _sc_scatter(vals, idx3, width):
    nh = 2 if width > D else 1
    half = CPT // nh
    mesh = plsc.VectorSubcoreMesh(core_axis_name="c", subcore_axis_name="s")

    def body(vals_hbm, idx_hbm, zeros_hbm, out_hbm, idx_v, vals_v, acc, sem):
        cid = lax.axis_index("c")
        sid = lax.axis_index("s")
        wid = sid * NC + cid
        pltpu.sync_copy(idx_hbm.at[wid], idx_v)
        pltpu.sync_copy(zeros_hbm.at[pl.ds(sid * NSTRIPE, NSTRIPE)],
                        acc.at[pl.ds(sid * NSTRIPE, NSTRIPE)])
        plsc.subcore_barrier()
        for hh in range(nh):
            pltpu.sync_copy(
                vals_hbm.at[pl.ds(wid * TPT + hh * half * CHUNK,
                                  half * CHUNK)], vals_v)
            copies = []
            for j in range(half):
                copies.append(pltpu.async_copy(
                    vals_v.at[pl.ds(j * CHUNK, CHUNK)],
                    acc.at[idx_v.at[hh * half + j]], sem, add=True))
            for c in copies:
                c.wait()
        plsc.subcore_barrier()
        pltpu.sync_copy(acc.at[pl.ds(sid * NSTRIPE, NSTRIPE)],
                        out_hbm.at[pl.ds(cid * N + sid * NSTRIPE, NSTRIPE)])

    f = pl.kernel(
        body,
        out_type=jax.ShapeDtypeStruct((NC * N, width), jnp.float32),
        mesh=mesh,
        compiler_params=pltpu.CompilerParams(use_tc_tiling_on_sc=False),
        scratch_types=[
            pltpu.VMEM((CPT, CHUNK), jnp.int32),
            pltpu.VMEM((half * CHUNK, width), jnp.float32),
            pltpu.VMEM_SHARED((N, width), jnp.float32),
            pltpu.SemaphoreType.DMA,
        ],
    )
    return f(vals, idx3, jnp.zeros((N, width), jnp.float32))


def _valid_col(i):
    rows = jax.lax.broadcasted_iota(jnp.int32, (EB, 1), 0)
    return (i * EB + rows < E).astype(jnp.float32)


def _edge1_body(ea_ref, xs_ref, w_ref, b_ref, r_ref, s_ref, out_ref):
    i = pl.program_id(0)
    valid = _valid_col(i)
    theta = jnp.dot(ea_ref[...], w_ref[...], preferred_element_type=jnp.float32)
    theta = jnp.maximum(theta + b_ref[...], 0.0)
    xs2 = jnp.dot(xs_ref[...], r_ref[...], preferred_element_type=jnp.float32)
    msg = jnp.dot(xs2 * theta, s_ref[...], preferred_element_type=jnp.float32)
    out_ref[:, :D] = msg * valid
    out_ref[:, D:] = jnp.broadcast_to(valid, (EB, D))


def _edge_body(ea_ref, xs_ref, w_ref, b_ref, r_ref, s_ref, out_ref):
    i = pl.program_id(0)
    valid = _valid_col(i)
    theta = jnp.dot(ea_ref[...], w_ref[...], preferred_element_type=jnp.float32)
    theta = jnp.maximum(theta + b_ref[...], 0.0)
    xs2 = jnp.dot(xs_ref[...], r_ref[...], preferred_element_type=jnp.float32)
    msg = jnp.dot(xs2 * theta, s_ref[...], preferred_element_type=jnp.float32)
    out_ref[...] = msg * valid


def _edge_msgs(ea_p, xs, nn_w, nn_b, with_ones):
    body = _edge1_body if with_ones else _edge_body
    width = 2 * D if with_ones else D
    return pl.pallas_call(
        body,
        grid=(GRID_EP,),
        in_specs=[
            pl.BlockSpec((EB, D), lambda i: (i, 0)),
            pl.BlockSpec((EB, D), lambda i: (i, 0)),
            pl.BlockSpec((D, D * D), lambda i: (0, 0)),
            pl.BlockSpec((1, D * D), lambda i: (0, 0)),
            pl.BlockSpec((D, D * D), lambda i: (0, 0)),
            pl.BlockSpec((D * D, D), lambda i: (0, 0)),
        ],
        out_specs=pl.BlockSpec((EB, width), lambda i: (i, 0)),
        out_shape=jax.ShapeDtypeStruct((EP, width), jnp.float32),
    )(ea_p, xs, nn_w, nn_b.reshape(1, -1), jnp.asarray(_R16),
      jnp.asarray(_S16))


def _node1_body(p_ref, x_ref, rw_ref, rb_ref, g_ref, b_ref, h_ref, cnt_ref):
    msum = p_ref[:N, :D] + p_ref[N:, :D]
    cnt = p_ref[:N, D:] + p_ref[N:, D:]
    agg = msum / jnp.maximum(cnt, 1.0)
    pre = agg + jnp.dot(x_ref[...], rw_ref[...],
                        preferred_element_type=jnp.float32) + rb_ref[...]
    mu = jnp.mean(pre, axis=0, keepdims=True)
    var = jnp.mean((pre - mu) * (pre - mu), axis=0, keepdims=True)
    h_ref[...] = (pre - mu) * jax.lax.rsqrt(var + 1e-5) * g_ref[...] + b_ref[...]
    cnt_ref[...] = cnt


def _node1_update(parts, x, root_w, root_b, g, b):
    return pl.pallas_call(
        _node1_body,
        out_shape=[jax.ShapeDtypeStruct((N, D), jnp.float32),
                   jax.ShapeDtypeStruct((N, D), jnp.float32)],
    )(parts, x, root_w, root_b.reshape(1, -1), g.reshape(1, -1),
      b.reshape(1, -1))


def _node_body(p_ref, c_ref, x_ref, rw_ref, rb_ref, g_ref, b_ref, h_ref):
    msum = p_ref[:N, :] + p_ref[N:, :]
    agg = msum / jnp.maximum(c_ref[...], 1.0)
    pre = agg + jnp.dot(x_ref[...], rw_ref[...],
                        preferred_element_type=jnp.float32) + rb_ref[...]
    mu = jnp.mean(pre, axis=0, keepdims=True)
    var = jnp.mean((pre - mu) * (pre - mu), axis=0, keepdims=True)
    h_ref[...] = (pre - mu) * jax.lax.rsqrt(var + 1e-5) * g_ref[...] + b_ref[...]


def _node_update(parts, cnt, x, root_w, root_b, g, b):
    return pl.pallas_call(
        _node_body,
        out_shape=jax.ShapeDtypeStruct((N, D), jnp.float32),
    )(parts, cnt, x, root_w, root_b.reshape(1, -1), g.reshape(1, -1),
      b.reshape(1, -1))


def _gat_edge_body(hs_ref, hd_ref, as_ref, ad_ref, out_ref):
    i = pl.program_id(0)
    valid = _valid_col(i)
    s = jnp.dot(hs_ref[...], as_ref[...], preferred_element_type=jnp.float32)
    t = jnp.dot(hd_ref[...], ad_ref[...], preferred_element_type=jnp.float32)
    e = s + t
    e = jnp.where(e >= 0, e, 0.2 * e)
    ex = jnp.exp(e) * valid
    out_ref[:, :D] = hs_ref[...] * ex
    out_ref[:, D:] = jnp.broadcast_to(ex, (EB, D))


def _gat_edge(ghw, a_src, a_dst):
    nblk = GRID_EP
    return pl.pallas_call(
        _gat_edge_body,
        grid=(nblk,),
        in_specs=[
            pl.BlockSpec((EB, D), lambda i: (i, 0)),
            pl.BlockSpec((EB, D), lambda i: (i + GRID_EP, 0)),
            pl.BlockSpec((D, 1), lambda i: (0, 0)),
            pl.BlockSpec((D, 1), lambda i: (0, 0)),
        ],
        out_specs=pl.BlockSpec((EB, 2 * D), lambda i: (i, 0)),
        out_shape=jax.ShapeDtypeStruct((EP, 2 * D), jnp.float32),
    )(ghw, ghw, a_src.reshape(-1, 1), a_dst.reshape(-1, 1))


def _gat_node_body(p_ref, bias_ref, g_ref, b_ref, h_ref):
    num = p_ref[:N, :D] + p_ref[N:, :D]
    den = p_ref[:N, D:] + p_ref[N:, D:]
    pre = num / jnp.maximum(den, 1e-16) + bias_ref[...]
    mu = jnp.mean(pre, axis=0, keepdims=True)
    var = jnp.mean((pre - mu) * (pre - mu), axis=0, keepdims=True)
    h_ref[...] = (pre - mu) * jax.lax.rsqrt(var + 1e-5) * g_ref[...] + b_ref[...]


def _gat_node(parts, bias, g, b):
    return pl.pallas_call(
        _gat_node_body,
        out_shape=jax.ShapeDtypeStruct((N, D), jnp.float32),
    )(parts, bias.reshape(1, -1), g.reshape(1, -1), b.reshape(1, -1))


def _matmul_body(x_ref, w_ref, o_ref):
    o_ref[...] = jnp.dot(x_ref[...], w_ref[...],
                         preferred_element_type=jnp.float32)


def _matmul(x, w):
    return pl.pallas_call(
        _matmul_body,
        out_shape=jax.ShapeDtypeStruct((x.shape[0], w.shape[1]), jnp.float32),
    )(x, w)


def _pool_body(hp_ref, bvp_ref, out_ref):
    hp = hp_ref[...]
    bvp = bvp_ref[...]
    neg = jnp.float32(-3.4e38)
    srows, crows, mrows = [], [], []
    for i in range(B):
        msk = bvp == i
        srows.append(jnp.sum(jnp.where(msk, hp, 0.0), axis=0, keepdims=True))
        crows.append(jnp.sum(jnp.where(msk, 1.0, 0.0), axis=0, keepdims=True))
        mrows.append(jnp.max(jnp.where(msk, hp, neg), axis=0, keepdims=True))
    sm = jnp.concatenate(srows, axis=0)
    cm = jnp.concatenate(crows, axis=0)
    mm = jnp.concatenate(mrows, axis=0)
    ssum = sm[:, 0:D]
    csum = cm[:, 0:D]
    mmax = mm[:, 0:D]
    for g in range(1, 8):
        ssum = ssum + sm[:, g * D:(g + 1) * D]
        csum = csum + cm[:, g * D:(g + 1) * D]
        mmax = jnp.maximum(mmax, mm[:, g * D:(g + 1) * D])
    cnt = csum
    mean = ssum / jnp.maximum(cnt, 1.0)
    mmax = jnp.where(cnt > 0, mmax, 0.0)
    out_ref[...] = jnp.concatenate([mean, mmax], axis=1)


def _pool(h, bv):
    hp = h.reshape(N // 8, 8 * D)
    bvp = jnp.repeat(bv, D).reshape(N // 8, 8 * D)
    return pl.pallas_call(
        _pool_body,
        out_shape=jax.ShapeDtypeStruct((B, 2 * D), jnp.float32),
    )(hp, bvp)


def kernel(x, edge_attr, c1_nn_w, c1_nn_b, c1_root_w, c1_root_b, c2_nn_w,
           c2_nn_b, c2_root_w, c2_root_b, c4_nn_w, c4_nn_b, c4_root_w,
           c4_root_b, bn1_g, bn1_b, bn2_g, bn2_b, bn3_g, bn3_b, bn4_g, bn4_b,
           gat_w, gat_asrc, gat_adst, gat_b, edge_index, batch_vector, mask):
    pad = jnp.zeros((EP - E,), jnp.int32)
    src3 = jnp.concatenate([edge_index[0], pad]).reshape(NW, CPT, CHUNK)
    dst3 = jnp.concatenate([edge_index[1], pad]).reshape(NW, CPT, CHUNK)
    idx6 = jnp.concatenate([src3, dst3], axis=1)
    ea_p = jnp.concatenate(
        [edge_attr, jnp.zeros((EP - E, D), jnp.float32)], axis=0)

    xs = _sc_gather(x, src3, 1)
    m1 = _edge_msgs(ea_p, xs, c1_nn_w, c1_nn_b, True)
    p1 = _sc_scatter(m1, dst3, 2 * D)
    h1, cnt = _node1_update(p1, x, c1_root_w, c1_root_b, bn1_g, bn1_b)

    hs1 = _sc_gather(h1, src3, 1)
    m2 = _edge_msgs(ea_p, hs1, c2_nn_w, c2_nn_b, False)
    p2 = _sc_scatter(m2, dst3, D)
    h2 = _node_update(p2, cnt, h1, c2_root_w, c2_root_b, bn2_g, bn2_b)

    hw = _matmul(h2, gat_w)
    ghw = _sc_gather(hw, idx6, 2)
    wex = _gat_edge(ghw, gat_asrc, gat_adst)
    p3 = _sc_scatter(wex, dst3, 2 * D)
    h3 = _gat_node(p3, gat_b, bn3_g, bn3_b)

    hs3 = _sc_gather(h3, src3, 1)
    m4 = _edge_msgs(ea_p, hs3, c4_nn_w, c4_nn_b, False)
    p4 = _sc_scatter(m4, dst3, D)
    h4 = _node_update(p4, cnt, h3, c4_root_w, c4_root_b, bn4_g, bn4_b)

    return _pool(h4, batch_vector)


# speedup vs baseline: 7.7952x; 1.5259x over previous
import functools
import jax
import jax.numpy as jnp
import numpy as np
from jax import lax
from jax.experimental import pallas as pl
from jax.experimental.pallas import tpu as pltpu
from jax.experimental.pallas import tpu_sc as plsc

N = 10000
E = 160000
D = 16
B = 64

NC = 2
NS = 16
NW = NC * NS
CHUNK = 128
CPT = 40
TPT = CPT * CHUNK
EP = NW * TPT
NSTRIPE = N // NS

EB = 1280
GRID_EP = EP // EB

_S16 = np.kron(np.ones((16, 1), np.float32), np.eye(16, dtype=np.float32))
_R_W = np.kron(np.eye(128, dtype=np.float32), np.ones((1, 16), np.float32))
_S_W = np.kron(np.eye(8, dtype=np.float32), _S16)
_I8 = np.eye(8, dtype=np.float32)
_ONES116 = np.ones((1, 16), np.float32)

EW = EB // 8
NVALID = E // 8


def _sc_gather(table, idx3, sections):
    scpt = sections * CPT
    mesh = plsc.VectorSubcoreMesh(core_axis_name="c", subcore_axis_name="s")

    def body(table_hbm, idx_hbm, out_hbm, idx_v, rows_v, sem):
        wid = lax.axis_index("s") * NC + lax.axis_index("c")
        pltpu.sync_copy(idx_hbm.at[wid], idx_v)
        for sec in range(sections):
            copies = []
            for j in range(CPT):
                copies.append(pltpu.async_copy(
                    table_hbm.at[idx_v.at[sec * CPT + j]],
                    rows_v.at[pl.ds(j * CHUNK, CHUNK)], sem))
            for c in copies:
                c.wait()
            pltpu.sync_copy(rows_v,
                            out_hbm.at[pl.ds(sec * EP + wid * TPT, TPT)])

    f = pl.kernel(
        body,
        out_type=jax.ShapeDtypeStruct((sections * EP, D), jnp.float32),
        mesh=mesh,
        compiler_params=pltpu.CompilerParams(use_tc_tiling_on_sc=False),
        scratch_types=[
            pltpu.VMEM((scpt, CHUNK), jnp.int32),
            pltpu.VMEM((TPT, D), jnp.float32),
            pltpu.SemaphoreType.DMA,
        ],
    )
    return f(table, idx3)


def _sc_scatter(vals, idx3, width):
    nh = 2 if width > D else 1
    half = CPT // nh
    mesh = plsc.VectorSubcoreMesh(core_axis_name="c", subcore_axis_name="s")

    def body(vals_hbm, idx_hbm, zeros_hbm, out_hbm, idx_v, vals_v, acc, sem):
        cid = lax.axis_index("c")
        sid = lax.axis_index("s")
        wid = sid * NC + cid
        pltpu.sync_copy(idx_hbm.at[wid], idx_v)
        pltpu.sync_copy(zeros_hbm.at[pl.ds(sid * NSTRIPE, NSTRIPE)],
                        acc.at[pl.ds(sid * NSTRIPE, NSTRIPE)])
        plsc.subcore_barrier()
        for hh in range(nh):
            pltpu.sync_copy(
                vals_hbm.at[pl.ds(wid * TPT + hh * half * CHUNK,
                                  half * CHUNK)], vals_v)
            copies = []
            for j in range(half):
                copies.append(pltpu.async_copy(
                    vals_v.at[pl.ds(j * CHUNK, CHUNK)],
                    acc.at[idx_v.at[hh * half + j]], sem, add=True))
            for c in copies:
                c.wait()
        plsc.subcore_barrier()
        pltpu.sync_copy(acc.at[pl.ds(sid * NSTRIPE, NSTRIPE)],
                        out_hbm.at[pl.ds(cid * N + sid * NSTRIPE, NSTRIPE)])

    f = pl.kernel(
        body,
        out_type=jax.ShapeDtypeStruct((NC * N, width), jnp.float32),
        mesh=mesh,
        compiler_params=pltpu.CompilerParams(use_tc_tiling_on_sc=False),
        scratch_types=[
            pltpu.VMEM((CPT, CHUNK), jnp.int32),
            pltpu.VMEM((half * CHUNK, width), jnp.float32),
            pltpu.VMEM_SHARED((N, width), jnp.float32),
            pltpu.SemaphoreType.DMA,
        ],
    )
    return f(vals, idx3, jnp.zeros((N, width), jnp.float32))


def _valid_col(i):
    rows = jax.lax.broadcasted_iota(jnp.int32, (EW, 1), 0)
    return (i * EW + rows < NVALID).astype(jnp.float32)


def _edge1_body(ea_ref, xs_ref, w_ref, b_ref, r_ref, s_ref, out_ref):
    valid = _valid_col(pl.program_id(0))
    theta = jnp.dot(ea_ref[...], w_ref[...], preferred_element_type=jnp.float32)
    theta = jnp.maximum(theta + b_ref[...], 0.0)
    xs2 = jnp.dot(xs_ref[...], r_ref[...], preferred_element_type=jnp.float32)
    msg = jnp.dot(xs2 * theta, s_ref[...],
                  preferred_element_type=jnp.float32) * valid
    vb = jnp.broadcast_to(valid, (EW, D))
    pieces = []
    for g in range(8):
        pieces.append(msg[:, g * D:(g + 1) * D])
        pieces.append(vb)
    out_ref[...] = jnp.concatenate(pieces, axis=1)


def _edge_body(ea_ref, xs_ref, w_ref, b_ref, r_ref, s_ref, out_ref):
    valid = _valid_col(pl.program_id(0))
    theta = jnp.dot(ea_ref[...], w_ref[...], preferred_element_type=jnp.float32)
    theta = jnp.maximum(theta + b_ref[...], 0.0)
    xs2 = jnp.dot(xs_ref[...], r_ref[...], preferred_element_type=jnp.float32)
    msg = jnp.dot(xs2 * theta, s_ref[...], preferred_element_type=jnp.float32)
    out_ref[...] = msg * valid


def _edge_msgs(ea_w, xs_w, nn_w, nn_b, with_ones):
    body = _edge1_body if with_ones else _edge_body
    width = 256 if with_ones else 128
    w_w = jnp.kron(jnp.asarray(_I8), nn_w)
    b_w = jnp.tile(nn_b, (8,)).reshape(1, -1)
    return pl.pallas_call(
        body,
        grid=(GRID_EP,),
        in_specs=[
            pl.BlockSpec((EW, 128), lambda i: (i, 0)),
            pl.BlockSpec((EW, 128), lambda i: (i, 0)),
            pl.BlockSpec((128, 2048), lambda i: (0, 0)),
            pl.BlockSpec((1, 2048), lambda i: (0, 0)),
            pl.BlockSpec((128, 2048), lambda i: (0, 0)),
            pl.BlockSpec((2048, 128), lambda i: (0, 0)),
        ],
        out_specs=pl.BlockSpec((EW, width), lambda i: (i, 0)),
        out_shape=jax.ShapeDtypeStruct((EP // 8, width), jnp.float32),
    )(ea_w, xs_w, w_w, b_w, jnp.asarray(_R_W), jnp.asarray(_S_W))


def _node1_body(p_ref, x_ref, rw_ref, rb_ref, g_ref, b_ref, h_ref, cnt_ref):
    msum = p_ref[:N, :D] + p_ref[N:, :D]
    cnt = p_ref[:N, D:] + p_ref[N:, D:]
    agg = msum / jnp.maximum(cnt, 1.0)
    pre = agg + jnp.dot(x_ref[...], rw_ref[...],
                        preferred_element_type=jnp.float32) + rb_ref[...]
    mu = jnp.mean(pre, axis=0, keepdims=True)
    var = jnp.mean((pre - mu) * (pre - mu), axis=0, keepdims=True)
    h_ref[...] = (pre - mu) * jax.lax.rsqrt(var + 1e-5) * g_ref[...] + b_ref[...]
    cnt_ref[...] = cnt


def _node1_update(parts, x, root_w, root_b, g, b):
    return pl.pallas_call(
        _node1_body,
        out_shape=[jax.ShapeDtypeStruct((N, D), jnp.float32),
                   jax.ShapeDtypeStruct((N, D), jnp.float32)],
    )(parts, x, root_w, root_b.reshape(1, -1), g.reshape(1, -1),
      b.reshape(1, -1))


def _node_body(p_ref, c_ref, x_ref, rw_ref, rb_ref, g_ref, b_ref, h_ref):
    msum = p_ref[:N, :] + p_ref[N:, :]
    agg = msum / jnp.maximum(c_ref[...], 1.0)
    pre = agg + jnp.dot(x_ref[...], rw_ref[...],
                        preferred_element_type=jnp.float32) + rb_ref[...]
    mu = jnp.mean(pre, axis=0, keepdims=True)
    var = jnp.mean((pre - mu) * (pre - mu), axis=0, keepdims=True)
    h_ref[...] = (pre - mu) * jax.lax.rsqrt(var + 1e-5) * g_ref[...] + b_ref[...]


def _node_update(parts, cnt, x, root_w, root_b, g, b):
    return pl.pallas_call(
        _node_body,
        out_shape=jax.ShapeDtypeStruct((N, D), jnp.float32),
    )(parts, cnt, x, root_w, root_b.reshape(1, -1), g.reshape(1, -1),
      b.reshape(1, -1))


def _gat_edge_body(hs_ref, hd_ref, as_ref, ad_ref, ob_ref, out_ref):
    valid = _valid_col(pl.program_id(0))
    s = jnp.dot(hs_ref[...], as_ref[...], preferred_element_type=jnp.float32)
    t = jnp.dot(hd_ref[...], ad_ref[...], preferred_element_type=jnp.float32)
    e = s + t
    e = jnp.where(e >= 0, e, 0.2 * e)
    ex = jnp.exp(e) * valid
    exb = jnp.dot(ex, ob_ref[...], preferred_element_type=jnp.float32)
    w = hs_ref[...] * exb
    pieces = []
    for g in range(8):
        pieces.append(w[:, g * D:(g + 1) * D])
        pieces.append(exb[:, g * D:(g + 1) * D])
    out_ref[...] = jnp.concatenate(pieces, axis=1)


def _gat_edge(ghw_w, a_src, a_dst):
    as_w = jnp.kron(jnp.asarray(_I8), a_src.reshape(-1, 1))
    ad_w = jnp.kron(jnp.asarray(_I8), a_dst.reshape(-1, 1))
    ones_w = jnp.kron(jnp.asarray(_I8), jnp.asarray(_ONES116))
    return pl.pallas_call(
        _gat_edge_body,
        grid=(GRID_EP,),
        in_specs=[
            pl.BlockSpec((EW, 128), lambda i: (i, 0)),
            pl.BlockSpec((EW, 128), lambda i: (i + GRID_EP, 0)),
            pl.BlockSpec((128, 8), lambda i: (0, 0)),
            pl.BlockSpec((128, 8), lambda i: (0, 0)),
            pl.BlockSpec((8, 128), lambda i: (0, 0)),
        ],
        out_specs=pl.BlockSpec((EW, 256), lambda i: (i, 0)),
        out_shape=jax.ShapeDtypeStruct((EP // 8, 256), jnp.float32),
    )(ghw_w, ghw_w, as_w, ad_w, ones_w)


def _gat_node_body(p_ref, bias_ref, g_ref, b_ref, h_ref):
    num = p_ref[:N, :D] + p_ref[N:, :D]
    den = p_ref[:N, D:] + p_ref[N:, D:]
    pre = num / jnp.maximum(den, 1e-16) + bias_ref[...]
    mu = jnp.mean(pre, axis=0, keepdims=True)
    var = jnp.mean((pre - mu) * (pre - mu), axis=0, keepdims=True)
    h_ref[...] = (pre - mu) * jax.lax.rsqrt(var + 1e-5) * g_ref[...] + b_ref[...]


def _gat_node(parts, bias, g, b):
    return pl.pallas_call(
        _gat_node_body,
        out_shape=jax.ShapeDtypeStruct((N, D), jnp.float32),
    )(parts, bias.reshape(1, -1), g.reshape(1, -1), b.reshape(1, -1))


def _matmul_body(x_ref, w_ref, o_ref):
    o_ref[...] = jnp.dot(x_ref[...], w_ref[...],
                         preferred_element_type=jnp.float32)


def _matmul(x, w):
    return pl.pallas_call(
        _matmul_body,
        out_shape=jax.ShapeDtypeStruct((x.shape[0], w.shape[1]), jnp.float32),
    )(x, w)


def _pool_body(hp_ref, bvp_ref, out_ref):
    hp = hp_ref[...]
    bvp = bvp_ref[...]
    neg = jnp.float32(-3.4e38)
    srows, crows, mrows = [], [], []
    for i in range(B):
        msk = bvp == i
        srows.append(jnp.sum(jnp.where(msk, hp, 0.0), axis=0, keepdims=True))
        crows.append(jnp.sum(jnp.where(msk, 1.0, 0.0), axis=0, keepdims=True))
        mrows.append(jnp.max(jnp.where(msk, hp, neg), axis=0, keepdims=True))
    sm = jnp.concatenate(srows, axis=0)
    cm = jnp.concatenate(crows, axis=0)
    mm = jnp.concatenate(mrows, axis=0)
    ssum = sm[:, 0:D]
    csum = cm[:, 0:D]
    mmax = mm[:, 0:D]
    for g in range(1, 8):
        ssum = ssum + sm[:, g * D:(g + 1) * D]
        csum = csum + cm[:, g * D:(g + 1) * D]
        mmax = jnp.maximum(mmax, mm[:, g * D:(g + 1) * D])
    cnt = csum
    mean = ssum / jnp.maximum(cnt, 1.0)
    mmax = jnp.where(cnt > 0, mmax, 0.0)
    out_ref[...] = jnp.concatenate([mean, mmax], axis=1)


def _pool(h, bv):
    hp = h.reshape(N // 8, 8 * D)
    bvp = jnp.repeat(bv, D).reshape(N // 8, 8 * D)
    return pl.pallas_call(
        _pool_body,
        out_shape=jax.ShapeDtypeStruct((B, 2 * D), jnp.float32),
    )(hp, bvp)


def kernel(x, edge_attr, c1_nn_w, c1_nn_b, c1_root_w, c1_root_b, c2_nn_w,
           c2_nn_b, c2_root_w, c2_root_b, c4_nn_w, c4_nn_b, c4_root_w,
           c4_root_b, bn1_g, bn1_b, bn2_g, bn2_b, bn3_g, bn3_b, bn4_g, bn4_b,
           gat_w, gat_asrc, gat_adst, gat_b, edge_index, batch_vector, mask):
    pad = jnp.zeros((EP - E,), jnp.int32)
    src3 = jnp.concatenate([edge_index[0], pad]).reshape(NW, CPT, CHUNK)
    dst3 = jnp.concatenate([edge_index[1], pad]).reshape(NW, CPT, CHUNK)
    idx6 = jnp.concatenate([src3, dst3], axis=1)
    ea_w = jnp.concatenate(
        [edge_attr.reshape(E // 8, 128),
         jnp.zeros(((EP - E) // 8, 128), jnp.float32)], axis=0)

    xs = _sc_gather(x, src3, 1).reshape(EP // 8, 128)
    m1 = _edge_msgs(ea_w, xs, c1_nn_w, c1_nn_b, True)
    p1 = _sc_scatter(m1.reshape(EP, 2 * D), dst3, 2 * D)
    h1, cnt = _node1_update(p1, x, c1_root_w, c1_root_b, bn1_g, bn1_b)

    hs1 = _sc_gather(h1, src3, 1).reshape(EP // 8, 128)
    m2 = _edge_msgs(ea_w, hs1, c2_nn_w, c2_nn_b, False)
    p2 = _sc_scatter(m2.reshape(EP, D), dst3, D)
    h2 = _node_update(p2, cnt, h1, c2_root_w, c2_root_b, bn2_g, bn2_b)

    hw = _matmul(h2, gat_w)
    ghw = _sc_gather(hw, idx6, 2).reshape(2 * EP // 8, 128)
    wex = _gat_edge(ghw, gat_asrc, gat_adst)
    p3 = _sc_scatter(wex.reshape(EP, 2 * D), dst3, 2 * D)
    h3 = _gat_node(p3, gat_b, bn3_g, bn3_b)

    hs3 = _sc_gather(h3, src3, 1).reshape(EP // 8, 128)
    m4 = _edge_msgs(ea_w, hs3, c4_nn_w, c4_nn_b, False)
    p4 = _sc_scatter(m4.reshape(EP, D), dst3, D)
    h4 = _node_update(p4, cnt, h3, c4_root_w, c4_root_b, bn4_g, bn4_b)

    return _pool(h4, batch_vector)


# speedup vs baseline: 8.2742x; 1.0615x over previous
import functools
import jax
import jax.numpy as jnp
import numpy as np
from jax import lax
from jax.experimental import pallas as pl
from jax.experimental.pallas import tpu as pltpu
from jax.experimental.pallas import tpu_sc as plsc

N = 10000
E = 160000
D = 16
B = 64

NC = 2
NS = 16
NW = NC * NS
CHUNK = 128
CPT = 40
TPT = CPT * CHUNK
EP = NW * TPT
NSTRIPE = N // NS

EB = 1280
GRID_EP = EP // EB

_S16 = np.kron(np.ones((16, 1), np.float32), np.eye(16, dtype=np.float32))
_R_W = np.kron(np.eye(128, dtype=np.float32), np.ones((1, 16), np.float32))
_S_W = np.kron(np.eye(8, dtype=np.float32), _S16)
_I8 = np.eye(8, dtype=np.float32)
_ONES116 = np.ones((1, 16), np.float32)

EW = EB // 8
NVALID = E // 8


def _sc_gather(table, idx3, sections):
    scpt = sections * CPT
    mesh = plsc.VectorSubcoreMesh(core_axis_name="c", subcore_axis_name="s")

    def body(table_hbm, idx_hbm, out_hbm, idx_v, rows_v, sem):
        wid = lax.axis_index("s") * NC + lax.axis_index("c")
        pltpu.sync_copy(idx_hbm.at[wid], idx_v)
        for sec in range(sections):
            copies = []
            for j in range(CPT):
                copies.append(pltpu.async_copy(
                    table_hbm.at[idx_v.at[sec * CPT + j]],
                    rows_v.at[pl.ds(j * CHUNK, CHUNK)], sem))
            for c in copies:
                c.wait()
            pltpu.sync_copy(rows_v,
                            out_hbm.at[pl.ds(sec * EP + wid * TPT, TPT)])

    f = pl.kernel(
        body,
        out_type=jax.ShapeDtypeStruct((sections * EP, D), jnp.float32),
        mesh=mesh,
        compiler_params=pltpu.CompilerParams(use_tc_tiling_on_sc=False),
        scratch_types=[
            pltpu.VMEM((scpt, CHUNK), jnp.int32),
            pltpu.VMEM((TPT, D), jnp.float32),
            pltpu.SemaphoreType.DMA,
        ],
    )
    return f(table, idx3)


def _sc_scatter(vals_w, idx3, width):
    nh = 2 if width > D else 1
    half = CPT // nh
    prow = N * width // 128
    pr10 = prow // 10
    vrow_h = half * CHUNK * width // 128
    mesh = plsc.VectorSubcoreMesh(core_axis_name="c", subcore_axis_name="s")

    def body(vals_hbm, idx_hbm, zeros_hbm, out_hbm, idx_v, vals_v, acc, sem):
        cid = lax.axis_index("c")
        sid = lax.axis_index("s")
        wid = sid * NC + cid
        pltpu.sync_copy(idx_hbm.at[wid], idx_v)
        pltpu.sync_copy(zeros_hbm.at[pl.ds(sid * NSTRIPE, NSTRIPE)],
                        acc.at[pl.ds(sid * NSTRIPE, NSTRIPE)])
        plsc.subcore_barrier()
        for hh in range(nh):
            pltpu.sync_copy(
                vals_hbm.at[pl.ds(wid * TPT + hh * half * CHUNK,
                                  half * CHUNK)], vals_v)
            copies = []
            for j in range(half):
                copies.append(pltpu.async_copy(
                    vals_v.at[pl.ds(j * CHUNK, CHUNK)],
                    acc.at[idx_v.at[hh * half + j]], sem, add=True))
            for c in copies:
                c.wait()
        plsc.subcore_barrier()
        pltpu.sync_copy(acc.at[pl.ds(sid * NSTRIPE, NSTRIPE)],
                        out_hbm.at[pl.ds(cid * N + sid * NSTRIPE, NSTRIPE)])

    f = pl.kernel(
        body,
        out_type=jax.ShapeDtypeStruct((NC * N, width), jnp.float32),
        mesh=mesh,
        compiler_params=pltpu.CompilerParams(use_tc_tiling_on_sc=False),
        scratch_types=[
            pltpu.VMEM((CPT, CHUNK), jnp.int32),
            pltpu.VMEM((half * CHUNK, width), jnp.float32),
            pltpu.VMEM_SHARED((N, width), jnp.float32),
            pltpu.SemaphoreType.DMA,
        ],
    )
    return f(vals_w, idx3, jnp.zeros((N, width), jnp.float32))


def _valid_col(i):
    rows = jax.lax.broadcasted_iota(jnp.int32, (EW, 1), 0)
    return (i * EW + rows < NVALID).astype(jnp.float32)


def _edge1_body(ea_ref, xs_ref, w_ref, b_ref, r_ref, s_ref, out_ref):
    valid = _valid_col(pl.program_id(0))
    theta = jnp.dot(ea_ref[...].astype(jnp.bfloat16), w_ref[...],
                    preferred_element_type=jnp.float32)
    theta = jnp.maximum(theta + b_ref[...], 0.0)
    xs2 = jnp.dot(xs_ref[...].astype(jnp.bfloat16), r_ref[...],
                  preferred_element_type=jnp.float32)
    msg = jnp.dot((xs2 * theta).astype(jnp.bfloat16), s_ref[...],
                  preferred_element_type=jnp.float32) * valid
    vb = jnp.broadcast_to(valid, (EW, D))
    pieces = []
    for g in range(8):
        pieces.append(msg[:, g * D:(g + 1) * D])
        pieces.append(vb)
    out_ref[...] = jnp.concatenate(pieces, axis=1)


def _edge_body(ea_ref, xs_ref, w_ref, b_ref, r_ref, s_ref, out_ref):
    valid = _valid_col(pl.program_id(0))
    theta = jnp.dot(ea_ref[...].astype(jnp.bfloat16), w_ref[...],
                    preferred_element_type=jnp.float32)
    theta = jnp.maximum(theta + b_ref[...], 0.0)
    xs2 = jnp.dot(xs_ref[...].astype(jnp.bfloat16), r_ref[...],
                  preferred_element_type=jnp.float32)
    msg = jnp.dot((xs2 * theta).astype(jnp.bfloat16), s_ref[...],
                  preferred_element_type=jnp.float32)
    out_ref[...] = msg * valid


def _edge_msgs(ea_w, xs_w, nn_w, nn_b, with_ones):
    body = _edge1_body if with_ones else _edge_body
    width = 256 if with_ones else 128
    w_w = jnp.kron(jnp.asarray(_I8), nn_w).astype(jnp.bfloat16)
    b_w = jnp.tile(nn_b, (8,)).reshape(1, -1)
    return pl.pallas_call(
        body,
        grid=(GRID_EP,),
        in_specs=[
            pl.BlockSpec((EW, 128), lambda i: (i, 0)),
            pl.BlockSpec((EW, 128), lambda i: (i, 0)),
            pl.BlockSpec((128, 2048), lambda i: (0, 0)),
            pl.BlockSpec((1, 2048), lambda i: (0, 0)),
            pl.BlockSpec((128, 2048), lambda i: (0, 0)),
            pl.BlockSpec((2048, 128), lambda i: (0, 0)),
        ],
        out_specs=pl.BlockSpec((EW, width), lambda i: (i, 0)),
        out_shape=jax.ShapeDtypeStruct((EP // 8, width), jnp.float32),
    )(ea_w, xs_w, w_w, b_w, jnp.asarray(_R_W, dtype=jnp.bfloat16),
      jnp.asarray(_S_W, dtype=jnp.bfloat16))


def _bn(pre, g_ref, b_ref):
    mu = jnp.mean(pre, axis=0, keepdims=True)
    var = jnp.mean((pre - mu) * (pre - mu), axis=0, keepdims=True)
    return (pre - mu) * jax.lax.rsqrt(var + 1e-5) * g_ref[...] + b_ref[...]


def _node1_body(p_ref, x_ref, rw_ref, rb_ref, g_ref, b_ref, h_ref, cnt_ref):
    sp = p_ref[:N, :] + p_ref[N:, :]
    msum = sp[:, :D]
    cnt = sp[:, D:]
    agg = msum / jnp.maximum(cnt, 1.0)
    pre = agg + jnp.dot(x_ref[...], rw_ref[...],
                        preferred_element_type=jnp.float32) + rb_ref[...]
    h_ref[...] = _bn(pre, g_ref, b_ref)
    cnt_ref[...] = cnt


def _node1_update(parts, x, root_w, root_b, g, b):
    return pl.pallas_call(
        _node1_body,
        out_shape=[jax.ShapeDtypeStruct((N, D), jnp.float32),
                   jax.ShapeDtypeStruct((N, D), jnp.float32)],
    )(parts, x, root_w, root_b.reshape(1, -1), g.reshape(1, -1),
      b.reshape(1, -1))


def _node_body(p_ref, c_ref, x_ref, rw_ref, rb_ref, g_ref, b_ref, h_ref):
    sp = p_ref[:N, :] + p_ref[N:, :]
    agg = sp / jnp.maximum(c_ref[...], 1.0)
    pre = agg + jnp.dot(x_ref[...], rw_ref[...],
                        preferred_element_type=jnp.float32) + rb_ref[...]
    h_ref[...] = _bn(pre, g_ref, b_ref)


def _node_update(parts, cnt, x, root_w, root_b, g, b):
    return pl.pallas_call(
        _node_body,
        out_shape=jax.ShapeDtypeStruct((N, D), jnp.float32),
    )(parts, cnt, x, root_w, root_b.reshape(1, -1), g.reshape(1, -1),
      b.reshape(1, -1))


def _matmul_body(x_ref, w_ref, o_ref):
    o_ref[...] = jnp.dot(x_ref[...], w_ref[...],
                         preferred_element_type=jnp.float32)


def _matmul(x, w):
    return pl.pallas_call(
        _matmul_body,
        out_shape=jax.ShapeDtypeStruct((x.shape[0], w.shape[1]), jnp.float32),
    )(x, w)


def _gat_edge_body(hs_ref, hd_ref, as_ref, ad_ref, ob_ref, out_ref):
    valid = _valid_col(pl.program_id(0))
    s = jnp.dot(hs_ref[...], as_ref[...], preferred_element_type=jnp.float32)
    t = jnp.dot(hd_ref[...], ad_ref[...], preferred_element_type=jnp.float32)
    e = s + t
    e = jnp.where(e >= 0, e, 0.2 * e)
    ex = jnp.exp(e) * valid
    exb = jnp.dot(ex, ob_ref[...], preferred_element_type=jnp.float32)
    w = hs_ref[...] * exb
    pieces = []
    for g in range(8):
        pieces.append(w[:, g * D:(g + 1) * D])
        pieces.append(exb[:, g * D:(g + 1) * D])
    out_ref[...] = jnp.concatenate(pieces, axis=1)


def _gat_edge(ghw_w, a_src, a_dst):
    as_w = jnp.kron(jnp.asarray(_I8), a_src.reshape(-1, 1))
    ad_w = jnp.kron(jnp.asarray(_I8), a_dst.reshape(-1, 1))
    ones_w = jnp.kron(jnp.asarray(_I8), jnp.asarray(_ONES116))
    return pl.pallas_call(
        _gat_edge_body,
        grid=(GRID_EP,),
        in_specs=[
            pl.BlockSpec((EW, 128), lambda i: (i, 0)),
            pl.BlockSpec((EW, 128), lambda i: (i + GRID_EP, 0)),
            pl.BlockSpec((128, 8), lambda i: (0, 0)),
            pl.BlockSpec((128, 8), lambda i: (0, 0)),
            pl.BlockSpec((8, 128), lambda i: (0, 0)),
        ],
        out_specs=pl.BlockSpec((EW, 256), lambda i: (i, 0)),
        out_shape=jax.ShapeDtypeStruct((EP // 8, 256), jnp.float32),
    )(ghw_w, ghw_w, as_w, ad_w, ones_w)


def _gat_node_body(p_ref, bias_ref, g_ref, b_ref, h_ref):
    sp = p_ref[:N, :] + p_ref[N:, :]
    num = sp[:, :D]
    den = sp[:, D:]
    pre = num / jnp.maximum(den, 1e-16) + bias_ref[...]
    h_ref[...] = _bn(pre, g_ref, b_ref)


def _gat_node(parts, bias, g, b):
    return pl.pallas_call(
        _gat_node_body,
        out_shape=jax.ShapeDtypeStruct((N, D), jnp.float32),
    )(parts, bias.reshape(1, -1), g.reshape(1, -1), b.reshape(1, -1))


def _pool_body(hp_ref, bvp_ref, out_ref):
    hp = hp_ref[...]
    bvp = bvp_ref[...]
    neg = jnp.float32(-3.4e38)
    srows, crows, mrows = [], [], []
    for i in range(B):
        msk = bvp == i
        srows.append(jnp.sum(jnp.where(msk, hp, 0.0), axis=0, keepdims=True))
        crows.append(jnp.sum(jnp.where(msk, 1.0, 0.0), axis=0, keepdims=True))
        mrows.append(jnp.max(jnp.where(msk, hp, neg), axis=0, keepdims=True))
    sm = jnp.concatenate(srows, axis=0)
    cm = jnp.concatenate(crows, axis=0)
    mm = jnp.concatenate(mrows, axis=0)
    ssum = sm[:, 0:D]
    csum = cm[:, 0:D]
    mmax = mm[:, 0:D]
    for g in range(1, 8):
        ssum = ssum + sm[:, g * D:(g + 1) * D]
        csum = csum + cm[:, g * D:(g + 1) * D]
        mmax = jnp.maximum(mmax, mm[:, g * D:(g + 1) * D])
    cnt = csum
    mean = ssum / jnp.maximum(cnt, 1.0)
    mmax = jnp.where(cnt > 0, mmax, 0.0)
    out_ref[...] = jnp.concatenate([mean, mmax], axis=1)


def _pool(h, bv):
    hp = h.reshape(N // 8, 8 * D)
    bvp = jnp.repeat(bv, D).reshape(N // 8, 8 * D)
    return pl.pallas_call(
        _pool_body,
        out_shape=jax.ShapeDtypeStruct((B, 2 * D), jnp.float32),
    )(hp, bvp)


def kernel(x, edge_attr, c1_nn_w, c1_nn_b, c1_root_w, c1_root_b, c2_nn_w,
           c2_nn_b, c2_root_w, c2_root_b, c4_nn_w, c4_nn_b, c4_root_w,
           c4_root_b, bn1_g, bn1_b, bn2_g, bn2_b, bn3_g, bn3_b, bn4_g, bn4_b,
           gat_w, gat_asrc, gat_adst, gat_b, edge_index, batch_vector, mask):
    pad = jnp.zeros((EP - E,), jnp.int32)
    src3 = jnp.concatenate([edge_index[0], pad]).reshape(NW, CPT, CHUNK)
    dst3 = jnp.concatenate([edge_index[1], pad]).reshape(NW, CPT, CHUNK)
    idx6 = jnp.concatenate([src3, dst3], axis=1)
    ea_w = jnp.concatenate(
        [edge_attr.reshape(E // 8, 128),
         jnp.zeros(((EP - E) // 8, 128), jnp.float32)], axis=0)

    xs = _sc_gather(x, src3, 1).reshape(EP // 8, 128)
    m1 = _edge_msgs(ea_w, xs, c1_nn_w, c1_nn_b, True)
    p1 = _sc_scatter(m1.reshape(EP, 2 * D), dst3, 2 * D)
    h1, cnt = _node1_update(p1, x, c1_root_w, c1_root_b, bn1_g, bn1_b)

    hs1 = _sc_gather(h1, src3, 1).reshape(EP // 8, 128)
    m2 = _edge_msgs(ea_w, hs1, c2_nn_w, c2_nn_b, False)
    p2 = _sc_scatter(m2.reshape(EP, D), dst3, D)
    h2 = _node_update(p2, cnt, h1, c2_root_w, c2_root_b, bn2_g, bn2_b)

    hw = _matmul(h2, gat_w)
    ghw = _sc_gather(hw, idx6, 2).reshape(2 * EP // 8, 128)
    wex = _gat_edge(ghw, gat_asrc, gat_adst)
    p3 = _sc_scatter(wex.reshape(EP, 2 * D), dst3, 2 * D)
    h3 = _gat_node(p3, gat_b, bn3_g, bn3_b)

    hs3 = _sc_gather(h3, src3, 1).reshape(EP // 8, 128)
    m4 = _edge_msgs(ea_w, hs3, c4_nn_w, c4_nn_b, False)
    p4 = _sc_scatter(m4.reshape(EP, D), dst3, D)
    h4 = _node_update(p4, cnt, h3, c4_root_w, c4_root_b, bn4_g, bn4_b)

    return _pool(h4, batch_vector)
